# baseline (speedup 1.0000x reference)
import functools

import jax
import jax.numpy as jnp
import numpy as np
from jax import lax
from jax.experimental import pallas as pl
from jax.experimental.pallas import tpu as pltpu
from jax.experimental.pallas import tpu_sc as plsc

B = 4096
NF = 26
V = 100000
M = 128
ROWS = B * NF
NW = 32
RPW = ROWS // NW
CHUNK = 128
NCH = RPW // CHUNK

BB = 256
GRID = B // BB
NI = 27
NP = 32


def _sc_gather(table2, gidx2):
    mesh = plsc.VectorSubcoreMesh(core_axis_name="c", subcore_axis_name="s")

    @functools.partial(
        pl.kernel,
        out_type=jax.ShapeDtypeStruct((ROWS, M), jnp.float32),
        mesh=mesh,
        scratch_types=[
            pltpu.VMEM((NCH, CHUNK), jnp.int32),
            pltpu.VMEM((CHUNK, M), jnp.float32),
            pltpu.SemaphoreType.DMA,
        ],
    )
    def k(table_hbm, gidx_hbm, out_hbm, idx_v, rows_v, sem):
        w = lax.axis_index("s") * 2 + lax.axis_index("c")
        pltpu.sync_copy(gidx_hbm.at[w], idx_v)

        def body(j, carry):
            pltpu.async_copy(table_hbm.at[idx_v.at[j]], rows_v, sem).wait()
            pltpu.sync_copy(rows_v, out_hbm.at[pl.ds(w * RPW + j * CHUNK, CHUNK)])
            return carry

        lax.fori_loop(0, NCH, body, 0)

    return k(table2, gidx2)


def _dott(a, w):
    return lax.dot_general(a, w, (((1,), (1,)), ((), ())),
                           preferred_element_type=jnp.float32)


def _tc_body(xd_ref, pooled_ref,
             wb0, bb0, wb1, bb1, wb2, bb2,
             wx, wz2, bt0, wt1, bt1, wt2, bt2, wt3, bt3, wt4, bt4,
             out_ref, tpad, zf3, zf2):
    h = jnp.maximum(_dott(xd_ref[...], wb0[...]) + bb0[...], 0.0)
    h = jnp.maximum(_dott(h, wb1[...]) + bb1[...], 0.0)
    xb = jnp.maximum(_dott(h, wb2[...]) + bb2[...], 0.0)

    tpad[:, 0:1, :] = xb[:, None, :]
    tpad[:, 1:NI, :] = pooled_ref[...]
    tpad[:, NI:NP, :] = jnp.zeros((BB, NP - NI, M), jnp.float32)

    for g in range(BB // 8):
        ts = tpad[g * 8:(g + 1) * 8].reshape(8 * NP, M).astype(jnp.bfloat16)
        zs = lax.dot_general(ts, ts, (((1,), (1,)), ((), ())),
                             preferred_element_type=jnp.float32)
        for s in range(8):
            zf3[g * 8 + s, :, :] = zs[s * NP:(s + 1) * NP, s * NP:(s + 1) * NP]

    for i in range(NP):
        zf2[:, i * NP:(i + 1) * NP] = zf3[:, i, :]

    l0 = jnp.maximum(
        _dott(xb.astype(jnp.bfloat16), wx[...])
        + _dott(zf2[...].astype(jnp.bfloat16), wz2[...]) + bt0[...], 0.0)
    l1 = jnp.maximum(_dott(l0.astype(jnp.bfloat16), wt1[...]) + bt1[...], 0.0)
    l2 = jnp.maximum(_dott(l1.astype(jnp.bfloat16), wt2[...]) + bt2[...], 0.0)
    l3 = jnp.maximum(_dott(l2, wt3[...]) + bt3[...], 0.0)
    l4 = _dott(l3, wt4[...])[:, 0:1]
    out_ref[...] = jax.nn.sigmoid(l4 + bt4[0, 0])


def _tc_call(dense_x, pooled3, args):
    full = lambda shape: pl.BlockSpec(shape, lambda i: (0,) * len(shape))
    in_specs = [
        pl.BlockSpec((BB, 13), lambda i: (i, 0)),
        pl.BlockSpec((BB, NF, M), lambda i: (i, 0, 0)),
    ] + [full(a.shape) for a in args[:-1]] + [
        pl.BlockSpec(memory_space=pltpu.SMEM)]
    return pl.pallas_call(
        _tc_body,
        grid=(GRID,),
        in_specs=in_specs,
        out_specs=pl.BlockSpec((BB, 1), lambda i: (i, 0)),
        out_shape=jax.ShapeDtypeStruct((B, 1), jnp.float32),
        scratch_shapes=[
            pltpu.VMEM((BB, NP, M), jnp.float32),
            pltpu.VMEM((BB, NP, NP), jnp.float32),
            pltpu.VMEM((BB, NP * NP), jnp.float32),
        ],
        compiler_params=pltpu.CompilerParams(
            dimension_semantics=("arbitrary",)),
    )(dense_x, pooled3, *args)


_LI = np.array([i for i in range(NI) for j in range(i)], dtype=np.int32)
_LJ = np.array([j for i in range(NI) for j in range(i)], dtype=np.int32)
_COLMAP = np.full(NP * NP, 351, dtype=np.int32)
_COLMAP[_LI * NP + _LJ] = np.arange(351, dtype=np.int32)


def kernel(dense_x, lS_o, lS_i, emb,
           W_bot0, b_bot0, W_bot1, b_bot1, W_bot2, b_bot2,
           W_top0, b_top0, W_top1, b_top1, W_top2, b_top2,
           W_top3, b_top3, W_top4, b_top4):
    gidx = (lS_i.T + (jnp.arange(NF, dtype=jnp.int32) * V)[None, :])
    gidx2 = gidx.reshape(NW, NCH, CHUNK)
    table2 = emb.reshape(NF * V, M)
    pooled3 = _sc_gather(table2, gidx2).reshape(B, NF, M)

    wsrc = jnp.concatenate(
        [W_top0[:, M:], jnp.zeros((W_top0.shape[0], 1), jnp.float32)], axis=1)
    wz2 = wsrc[:, _COLMAP]
    wx = W_top0[:, :M]

    args = [
        W_bot0, b_bot0.reshape(1, -1), W_bot1, b_bot1.reshape(1, -1),
        W_bot2, b_bot2.reshape(1, -1),
        wx.astype(jnp.bfloat16), wz2.astype(jnp.bfloat16), b_top0.reshape(1, -1),
        W_top1.astype(jnp.bfloat16), b_top1.reshape(1, -1),
        W_top2.astype(jnp.bfloat16), b_top2.reshape(1, -1),
        W_top3, b_top3.reshape(1, -1),
        jnp.concatenate([W_top4, jnp.zeros((7, W_top4.shape[1]), jnp.float32)]),
        b_top4.reshape(1, -1),
    ]
    return _tc_call(dense_x, pooled3, args)


# speedup vs baseline: 1.0758x; 1.0758x over previous
import functools

import jax
import jax.numpy as jnp
import numpy as np
from jax import lax
from jax.experimental import pallas as pl
from jax.experimental.pallas import tpu as pltpu
from jax.experimental.pallas import tpu_sc as plsc

B = 4096
NF = 26
V = 100000
M = 128
ROWS = B * NF
NW = 32
RPW = ROWS // NW
CHUNK = 128
NCH = RPW // CHUNK

BB = 256
GRID = B // BB
NI = 27
NP = 32


def _sc_gather(table2, gidx2):
    mesh = plsc.VectorSubcoreMesh(core_axis_name="c", subcore_axis_name="s")

    @functools.partial(
        pl.kernel,
        out_type=jax.ShapeDtypeStruct((ROWS, M), jnp.float32),
        mesh=mesh,
        scratch_types=[
            pltpu.VMEM((NCH, CHUNK), jnp.int32),
            pltpu.VMEM((CHUNK, M), jnp.float32),
            pltpu.SemaphoreType.DMA,
        ],
    )
    def k(table_hbm, gidx_hbm, out_hbm, idx_v, rows_v, sem):
        w = lax.axis_index("s") * 2 + lax.axis_index("c")
        pltpu.sync_copy(gidx_hbm.at[w], idx_v)

        def body(j, carry):
            pltpu.async_copy(table_hbm.at[idx_v.at[j]], rows_v, sem).wait()
            pltpu.sync_copy(rows_v, out_hbm.at[pl.ds(w * RPW + j * CHUNK, CHUNK)])
            return carry

        lax.fori_loop(0, NCH, body, 0)

    return k(table2, gidx2)


def _dott(a, w):
    return lax.dot_general(a, w, (((1,), (1,)), ((), ())),
                           preferred_element_type=jnp.float32)


def _tc_body(xd_ref, pooled_ref,
             wb0, bb0, wb1, bb1, wb2, bb2,
             wx, wz2, bt0, wt1, bt1, wt2, bt2, wt3, bt3, wt4, bt4,
             out_ref, tpad, zf3):
    h = jnp.maximum(_dott(xd_ref[...], wb0[...]) + bb0[...], 0.0)
    h = jnp.maximum(_dott(h, wb1[...]) + bb1[...], 0.0)
    xb = jnp.maximum(_dott(h, wb2[...]) + bb2[...], 0.0)

    tpad[:, 0:1, :] = xb[:, None, :]
    tpad[:, 1:NI, :] = pooled_ref[...]
    tpad[:, NI:NP, :] = jnp.zeros((BB, NP - NI, M), jnp.float32)

    for g in range(BB // 8):
        ts = tpad[g * 8:(g + 1) * 8].reshape(8 * NP, M).astype(jnp.bfloat16)
        zs = lax.dot_general(ts, ts, (((1,), (1,)), ((), ())),
                             preferred_element_type=jnp.float32)
        for s in range(8):
            zf3[g * 8 + s, :, :] = zs[s * NP:(s + 1) * NP,
                                      s * NP:(s + 1) * NP].astype(jnp.bfloat16)

    zint = jnp.zeros((BB, 1024), jnp.float32)
    for q in range(NP // 4):
        zq = zf3[:, 4 * q:4 * (q + 1), :].reshape(BB, 4 * NP)
        zint = zint + _dott(zq, wz2[q])

    l0 = jnp.maximum(
        _dott(xb.astype(jnp.bfloat16), wx[...]) + zint + bt0[...], 0.0)
    l1 = jnp.maximum(_dott(l0.astype(jnp.bfloat16), wt1[...]) + bt1[...], 0.0)
    l2 = jnp.maximum(_dott(l1.astype(jnp.bfloat16), wt2[...]) + bt2[...], 0.0)
    l3 = jnp.maximum(_dott(l2, wt3[...]) + bt3[...], 0.0)
    l4 = _dott(l3, wt4[...])[:, 0:1]
    out_ref[...] = jax.nn.sigmoid(l4 + bt4[0, 0])


def _tc_call(dense_x, pooled3, args):
    full = lambda shape: pl.BlockSpec(shape, lambda i: (0,) * len(shape))
    in_specs = [
        pl.BlockSpec((BB, 13), lambda i: (i, 0)),
        pl.BlockSpec((BB, NF, M), lambda i: (i, 0, 0)),
    ] + [full(a.shape) for a in args[:-1]] + [
        pl.BlockSpec(memory_space=pltpu.SMEM)]
    return pl.pallas_call(
        _tc_body,
        grid=(GRID,),
        in_specs=in_specs,
        out_specs=pl.BlockSpec((BB, 1), lambda i: (i, 0)),
        out_shape=jax.ShapeDtypeStruct((B, 1), jnp.float32),
        scratch_shapes=[
            pltpu.VMEM((BB, NP, M), jnp.float32),
            pltpu.VMEM((BB, NP, NP), jnp.bfloat16),
        ],
        compiler_params=pltpu.CompilerParams(
            dimension_semantics=("arbitrary",)),
    )(dense_x, pooled3, *args)


_LI = np.array([i for i in range(NI) for j in range(i)], dtype=np.int32)
_LJ = np.array([j for i in range(NI) for j in range(i)], dtype=np.int32)
_COLMAP = np.full(NP * NP, 351, dtype=np.int32)
_COLMAP[_LI * NP + _LJ] = np.arange(351, dtype=np.int32)


def kernel(dense_x, lS_o, lS_i, emb,
           W_bot0, b_bot0, W_bot1, b_bot1, W_bot2, b_bot2,
           W_top0, b_top0, W_top1, b_top1, W_top2, b_top2,
           W_top3, b_top3, W_top4, b_top4):
    gidx = (lS_i.T + (jnp.arange(NF, dtype=jnp.int32) * V)[None, :])
    gidx2 = gidx.reshape(NW, NCH, CHUNK)
    table2 = emb.reshape(NF * V, M)
    pooled3 = _sc_gather(table2, gidx2).reshape(B, NF, M)

    wsrc = jnp.concatenate(
        [W_top0[:, M:], jnp.zeros((W_top0.shape[0], 1), jnp.float32)], axis=1)
    wz2 = wsrc[:, _COLMAP]
    wx = W_top0[:, :M]

    args = [
        W_bot0, b_bot0.reshape(1, -1), W_bot1, b_bot1.reshape(1, -1),
        W_bot2, b_bot2.reshape(1, -1),
        wx.astype(jnp.bfloat16),
        wz2.astype(jnp.bfloat16).reshape(-1, 8, 4 * NP).transpose(1, 0, 2),
        b_top0.reshape(1, -1),
        W_top1.astype(jnp.bfloat16), b_top1.reshape(1, -1),
        W_top2.astype(jnp.bfloat16), b_top2.reshape(1, -1),
        W_top3, b_top3.reshape(1, -1),
        jnp.concatenate([W_top4, jnp.zeros((7, W_top4.shape[1]), jnp.float32)]),
        b_top4.reshape(1, -1),
    ]
    return _tc_call(dense_x, pooled3, args)


# speedup vs baseline: 1.4637x; 1.3606x over previous
import functools

import jax
import jax.numpy as jnp
import numpy as np
from jax import lax
from jax.experimental import pallas as pl
from jax.experimental.pallas import tpu as pltpu
from jax.experimental.pallas import tpu_sc as plsc

B = 4096
NF = 26
V = 100000
M = 128
NW = 32
CHUNK = 128

BB = 256
GRID = B // BB
NI = 27
NP = 32


def _sc_gather(table2, gidx3):
    mesh = plsc.VectorSubcoreMesh(core_axis_name="c", subcore_axis_name="s")

    @functools.partial(
        pl.kernel,
        out_type=jax.ShapeDtypeStruct((B, NP, M), jnp.float32),
        mesh=mesh,
        scratch_types=[
            pltpu.VMEM((NF, CHUNK), jnp.int32),
            pltpu.VMEM((CHUNK, 1, M), jnp.float32),
            pltpu.SemaphoreType.DMA,
        ],
    )
    def k(table_hbm, gidx_hbm, out_hbm, idx_v, rows_v, sem):
        w = lax.axis_index("s") * 2 + lax.axis_index("c")
        b0 = w * CHUNK
        pltpu.sync_copy(gidx_hbm.at[w], idx_v)

        def body(kf, carry):
            pltpu.async_copy(table_hbm.at[idx_v.at[kf]], rows_v.at[:, 0], sem).wait()
            pltpu.sync_copy(rows_v,
                            out_hbm.at[pl.ds(b0, CHUNK), pl.ds(kf + 1, 1)])
            return carry

        lax.fori_loop(0, NF, body, 0)

    return k(table2, gidx3)


def _dott(a, w):
    return lax.dot_general(a, w, (((1,), (1,)), ((), ())),
                           preferred_element_type=jnp.float32)


def _tc_body(xd_ref, pooled_ref,
             wb0, bb0, wb1, bb1, wb2, bb2,
             wx, wz2, bt0, wt1, bt1, wt2, bt2, wt3, bt3, wt4, bt4,
             out_ref, tpad, zf3):
    h = jnp.maximum(_dott(xd_ref[...], wb0[...]) + bb0[...], 0.0)
    h = jnp.maximum(_dott(h, wb1[...]) + bb1[...], 0.0)
    xb = jnp.maximum(_dott(h, wb2[...]) + bb2[...], 0.0)

    tpad[...] = pooled_ref[...]
    tpad[:, 0:1, :] = xb[:, None, :]
    tpad[:, NI:NP, :] = jnp.zeros((BB, NP - NI, M), jnp.float32)

    for g in range(BB // 8):
        ts = tpad[g * 8:(g + 1) * 8].reshape(8 * NP, M).astype(jnp.bfloat16)
        zs = lax.dot_general(ts, ts, (((1,), (1,)), ((), ())),
                             preferred_element_type=jnp.float32)
        for s in range(8):
            zf3[g * 8 + s, :, :] = zs[s * NP:(s + 1) * NP,
                                      s * NP:(s + 1) * NP].astype(jnp.bfloat16)

    zint = jnp.zeros((BB, 1024), jnp.float32)
    for q in range(NP // 4):
        zq = zf3[:, 4 * q:4 * (q + 1), :].reshape(BB, 4 * NP)
        zint = zint + _dott(zq, wz2[q])

    l0 = jnp.maximum(
        _dott(xb.astype(jnp.bfloat16), wx[...]) + zint + bt0[...], 0.0)
    l1 = jnp.maximum(_dott(l0.astype(jnp.bfloat16), wt1[...]) + bt1[...], 0.0)
    l2 = jnp.maximum(_dott(l1.astype(jnp.bfloat16), wt2[...]) + bt2[...], 0.0)
    l3 = jnp.maximum(_dott(l2, wt3[...]) + bt3[...], 0.0)
    l4 = _dott(l3, wt4[...])[:, 0:1]
    out_ref[...] = jax.nn.sigmoid(l4 + bt4[0, 0])


def _tc_call(dense_x, pooled3, args):
    full = lambda shape: pl.BlockSpec(shape, lambda i: (0,) * len(shape))
    in_specs = [
        pl.BlockSpec((BB, 13), lambda i: (i, 0)),
        pl.BlockSpec((BB, NP, M), lambda i: (i, 0, 0)),
    ] + [full(a.shape) for a in args[:-1]] + [
        pl.BlockSpec(memory_space=pltpu.SMEM)]
    return pl.pallas_call(
        _tc_body,
        grid=(GRID,),
        in_specs=in_specs,
        out_specs=pl.BlockSpec((BB, 1), lambda i: (i, 0)),
        out_shape=jax.ShapeDtypeStruct((B, 1), jnp.float32),
        scratch_shapes=[
            pltpu.VMEM((BB, NP, M), jnp.float32),
            pltpu.VMEM((BB, NP, NP), jnp.bfloat16),
        ],
        compiler_params=pltpu.CompilerParams(
            dimension_semantics=("arbitrary",)),
    )(dense_x, pooled3, *args)


_LI = np.array([i for i in range(NI) for j in range(i)], dtype=np.int32)
_LJ = np.array([j for i in range(NI) for j in range(i)], dtype=np.int32)
_COLMAP = np.full(NP * NP, 351, dtype=np.int32)
_COLMAP[_LI * NP + _LJ] = np.arange(351, dtype=np.int32)


def kernel(dense_x, lS_o, lS_i, emb,
           W_bot0, b_bot0, W_bot1, b_bot1, W_bot2, b_bot2,
           W_top0, b_top0, W_top1, b_top1, W_top2, b_top2,
           W_top3, b_top3, W_top4, b_top4):
    gidx = lS_i + (jnp.arange(NF, dtype=jnp.int32) * V)[:, None]
    gidx3 = gidx.reshape(NF, NW, CHUNK).transpose(1, 0, 2)
    table2 = emb.reshape(NF * V, M)
    pooled3 = _sc_gather(table2, gidx3)

    wsrc = jnp.concatenate(
        [W_top0[:, M:], jnp.zeros((W_top0.shape[0], 1), jnp.float32)], axis=1)
    wz2 = wsrc[:, _COLMAP]
    wx = W_top0[:, :M]

    args = [
        W_bot0, b_bot0.reshape(1, -1), W_bot1, b_bot1.reshape(1, -1),
        W_bot2, b_bot2.reshape(1, -1),
        wx.astype(jnp.bfloat16),
        wz2.astype(jnp.bfloat16).reshape(-1, 8, 4 * NP).transpose(1, 0, 2),
        b_top0.reshape(1, -1),
        W_top1.astype(jnp.bfloat16), b_top1.reshape(1, -1),
        W_top2.astype(jnp.bfloat16), b_top2.reshape(1, -1),
        W_top3, b_top3.reshape(1, -1),
        jnp.concatenate([W_top4, jnp.zeros((7, W_top4.shape[1]), jnp.float32)]),
        b_top4.reshape(1, -1),
    ]
    return _tc_call(dense_x, pooled3, args)


# speedup vs baseline: 1.5536x; 1.0614x over previous
import functools

import jax
import jax.numpy as jnp
import numpy as np
from jax import lax
from jax.experimental import pallas as pl
from jax.experimental.pallas import tpu as pltpu
from jax.experimental.pallas import tpu_sc as plsc

B = 4096
NF = 26
V = 100000
M = 128
NW = 32
CHUNK = 128

BB = 512
GRID = B // BB
NI = 27
NP = 32


def _sc_gather(table2, gidx3):
    mesh = plsc.VectorSubcoreMesh(core_axis_name="c", subcore_axis_name="s")

    @functools.partial(
        pl.kernel,
        out_type=jax.ShapeDtypeStruct((B, NP, M), jnp.float32),
        mesh=mesh,
        scratch_types=[
            pltpu.VMEM((NF, CHUNK), jnp.int32),
            pltpu.VMEM((CHUNK, 1, M), jnp.float32),
            pltpu.SemaphoreType.DMA,
        ],
    )
    def k(table_hbm, gidx_hbm, out_hbm, idx_v, rows_v, sem):
        w = lax.axis_index("s") * 2 + lax.axis_index("c")
        b0 = w * CHUNK
        pltpu.sync_copy(gidx_hbm.at[w], idx_v)

        def body(kf, carry):
            pltpu.async_copy(table_hbm.at[idx_v.at[kf]], rows_v.at[:, 0], sem).wait()
            pltpu.sync_copy(rows_v,
                            out_hbm.at[pl.ds(b0, CHUNK), pl.ds(kf + 1, 1)])
            return carry

        lax.fori_loop(0, NF, body, 0)

    return k(table2, gidx3)


def _dott(a, w):
    return lax.dot_general(a, w, (((1,), (1,)), ((), ())),
                           preferred_element_type=jnp.float32)


def _tc_body(xd_ref, pooled_ref,
             wb0, bb0, wb1, bb1, wb2, bb2,
             wx, wz2, bt0, wt1, bt1, wt2, bt2, wt3, bt3, wt4, bt4,
             out_ref, tpad, zf3):
    h = jnp.maximum(_dott(xd_ref[...], wb0[...]) + bb0[...], 0.0)
    h = jnp.maximum(_dott(h, wb1[...]) + bb1[...], 0.0)
    xb = jnp.maximum(_dott(h, wb2[...]) + bb2[...], 0.0)

    tpad[...] = pooled_ref[...]
    tpad[:, 0:1, :] = xb[:, None, :]
    tpad[:, NI:NP, :] = jnp.zeros((BB, NP - NI, M), jnp.float32)

    for g in range(BB // 8):
        ts = tpad[g * 8:(g + 1) * 8].reshape(8 * NP, M).astype(jnp.bfloat16)
        zs = lax.dot_general(ts, ts, (((1,), (1,)), ((), ())),
                             preferred_element_type=jnp.float32)
        for s in range(8):
            zf3[g * 8 + s, :, :] = zs[s * NP:(s + 1) * NP,
                                      s * NP:(s + 1) * NP].astype(jnp.bfloat16)

    zint = jnp.zeros((BB, 1024), jnp.float32)
    for q in range(NP // 4):
        zq = zf3[:, 4 * q:4 * (q + 1), :].reshape(BB, 4 * NP)
        zint = zint + _dott(zq, wz2[q])

    l0 = jnp.maximum(
        _dott(xb.astype(jnp.bfloat16), wx[...]) + zint + bt0[...], 0.0)
    l1 = jnp.maximum(_dott(l0.astype(jnp.bfloat16), wt1[...]) + bt1[...], 0.0)
    l2 = jnp.maximum(_dott(l1.astype(jnp.bfloat16), wt2[...]) + bt2[...], 0.0)
    l3 = jnp.maximum(_dott(l2, wt3[...]) + bt3[...], 0.0)
    l4 = _dott(l3, wt4[...])[:, 0:1]
    out_ref[...] = jax.nn.sigmoid(l4 + bt4[0, 0])


def _tc_call(dense_x, pooled3, args):
    full = lambda shape: pl.BlockSpec(shape, lambda i: (0,) * len(shape))
    in_specs = [
        pl.BlockSpec((BB, 13), lambda i: (i, 0)),
        pl.BlockSpec((BB, NP, M), lambda i: (i, 0, 0)),
    ] + [full(a.shape) for a in args[:-1]] + [
        pl.BlockSpec(memory_space=pltpu.SMEM)]
    return pl.pallas_call(
        _tc_body,
        grid=(GRID,),
        in_specs=in_specs,
        out_specs=pl.BlockSpec((BB, 1), lambda i: (i, 0)),
        out_shape=jax.ShapeDtypeStruct((B, 1), jnp.float32),
        scratch_shapes=[
            pltpu.VMEM((BB, NP, M), jnp.float32),
            pltpu.VMEM((BB, NP, NP), jnp.bfloat16),
        ],
        compiler_params=pltpu.CompilerParams(
            dimension_semantics=("arbitrary",)),
    )(dense_x, pooled3, *args)


_LI = np.array([i for i in range(NI) for j in range(i)], dtype=np.int32)
_LJ = np.array([j for i in range(NI) for j in range(i)], dtype=np.int32)
_COLMAP = np.full(NP * NP, 351, dtype=np.int32)
_COLMAP[_LI * NP + _LJ] = np.arange(351, dtype=np.int32)


def kernel(dense_x, lS_o, lS_i, emb,
           W_bot0, b_bot0, W_bot1, b_bot1, W_bot2, b_bot2,
           W_top0, b_top0, W_top1, b_top1, W_top2, b_top2,
           W_top3, b_top3, W_top4, b_top4):
    gidx = lS_i + (jnp.arange(NF, dtype=jnp.int32) * V)[:, None]
    gidx3 = gidx.reshape(NF, NW, CHUNK).transpose(1, 0, 2)
    table2 = emb.reshape(NF * V, M)
    pooled3 = _sc_gather(table2, gidx3)

    wsrc = jnp.concatenate(
        [W_top0[:, M:], jnp.zeros((W_top0.shape[0], 1), jnp.float32)], axis=1)
    wz2 = wsrc[:, _COLMAP]
    wx = W_top0[:, :M]

    args = [
        W_bot0, b_bot0.reshape(1, -1), W_bot1, b_bot1.reshape(1, -1),
        W_bot2, b_bot2.reshape(1, -1),
        wx.astype(jnp.bfloat16),
        wz2.astype(jnp.bfloat16).reshape(-1, 8, 4 * NP).transpose(1, 0, 2),
        b_top0.reshape(1, -1),
        W_top1.astype(jnp.bfloat16), b_top1.reshape(1, -1),
        W_top2.astype(jnp.bfloat16), b_top2.reshape(1, -1),
        W_top3, b_top3.reshape(1, -1),
        jnp.concatenate([W_top4, jnp.zeros((7, W_top4.shape[1]), jnp.float32)]),
        b_top4.reshape(1, -1),
    ]
    return _tc_call(dense_x, pooled3, args)


# speedup vs baseline: 1.5814x; 1.0179x over previous
import functools

import jax
import jax.numpy as jnp
import numpy as np
from jax import lax
from jax.experimental import pallas as pl
from jax.experimental.pallas import tpu as pltpu
from jax.experimental.pallas import tpu_sc as plsc

B = 4096
NF = 26
V = 100000
M = 128
NW = 32
CHUNK = 64

NSPLIT = 2
BSPLIT = B // NSPLIT
BB = 512
GRID = BSPLIT // BB
NI = 27
NP = 32


def _sc_gather(table2, gidx3):
    mesh = plsc.VectorSubcoreMesh(core_axis_name="c", subcore_axis_name="s")

    @functools.partial(
        pl.kernel,
        out_type=jax.ShapeDtypeStruct((BSPLIT, NP, M), jnp.float32),
        mesh=mesh,
        scratch_types=[
            pltpu.VMEM((NF, CHUNK), jnp.int32),
            pltpu.VMEM((CHUNK, 1, M), jnp.float32),
            pltpu.SemaphoreType.DMA,
        ],
    )
    def k(table_hbm, gidx_hbm, out_hbm, idx_v, rows_v, sem):
        w = lax.axis_index("s") * 2 + lax.axis_index("c")
        b0 = w * CHUNK
        pltpu.sync_copy(gidx_hbm.at[w], idx_v)

        def body(kf, carry):
            pltpu.async_copy(table_hbm.at[idx_v.at[kf]], rows_v.at[:, 0], sem).wait()
            pltpu.sync_copy(rows_v,
                            out_hbm.at[pl.ds(b0, CHUNK), pl.ds(kf + 1, 1)])
            return carry

        lax.fori_loop(0, NF, body, 0)

    return k(table2, gidx3)


def _dott(a, w):
    return lax.dot_general(a, w, (((1,), (1,)), ((), ())),
                           preferred_element_type=jnp.float32)


def _tc_body(xd_ref, pooled_ref,
             wb0, bb0, wb1, bb1, wb2, bb2,
             wx, wz2, bt0, wt1, bt1, wt2, bt2, wt3, bt3, wt4, bt4,
             out_ref, tpad, zf3):
    h = jnp.maximum(_dott(xd_ref[...], wb0[...]) + bb0[...], 0.0)
    h = jnp.maximum(_dott(h, wb1[...]) + bb1[...], 0.0)
    xb = jnp.maximum(_dott(h, wb2[...]) + bb2[...], 0.0)

    tpad[...] = pooled_ref[...].astype(jnp.bfloat16)
    tpad[:, 0:1, :] = xb[:, None, :].astype(jnp.bfloat16)
    tpad[:, NI:NP, :] = jnp.zeros((BB, NP - NI, M), jnp.bfloat16)

    for g in range(BB // 8):
        ts = tpad[g * 8:(g + 1) * 8].reshape(8 * NP, M)
        zs = lax.dot_general(ts, ts, (((1,), (1,)), ((), ())),
                             preferred_element_type=jnp.float32)
        for s in range(8):
            zf3[g * 8 + s, :, :] = zs[s * NP:(s + 1) * NP,
                                      s * NP:(s + 1) * NP].astype(jnp.bfloat16)

    zint = jnp.zeros((BB, 1024), jnp.float32)
    for q in range(NP // 4):
        zq = zf3[:, 4 * q:4 * (q + 1), :].reshape(BB, 4 * NP)
        zint = zint + _dott(zq, wz2[q])

    l0 = jnp.maximum(
        _dott(xb.astype(jnp.bfloat16), wx[...]) + zint + bt0[...], 0.0)
    l1 = jnp.maximum(_dott(l0.astype(jnp.bfloat16), wt1[...]) + bt1[...], 0.0)
    l2 = jnp.maximum(_dott(l1.astype(jnp.bfloat16), wt2[...]) + bt2[...], 0.0)
    l3 = jnp.maximum(_dott(l2, wt3[...]) + bt3[...], 0.0)
    l4 = _dott(l3, wt4[...])[:, 0:1]
    out_ref[...] = jax.nn.sigmoid(l4 + bt4[0, 0])


def _tc_call(dense_x, pooled3, args):
    full = lambda shape: pl.BlockSpec(shape, lambda i: (0,) * len(shape))
    in_specs = [
        pl.BlockSpec((BB, 13), lambda i: (i, 0)),
        pl.BlockSpec((BB, NP, M), lambda i: (i, 0, 0)),
    ] + [full(a.shape) for a in args[:-1]] + [
        pl.BlockSpec(memory_space=pltpu.SMEM)]
    return pl.pallas_call(
        _tc_body,
        grid=(GRID,),
        in_specs=in_specs,
        out_specs=pl.BlockSpec((BB, 1), lambda i: (i, 0)),
        out_shape=jax.ShapeDtypeStruct((B, 1), jnp.float32),
        scratch_shapes=[
            pltpu.VMEM((BB, NP, M), jnp.bfloat16),
            pltpu.VMEM((BB, NP, NP), jnp.bfloat16),
        ],
        compiler_params=pltpu.CompilerParams(
            dimension_semantics=("arbitrary",)),
    )(dense_x, pooled3, *args)


_LI = np.array([i for i in range(NI) for j in range(i)], dtype=np.int32)
_LJ = np.array([j for i in range(NI) for j in range(i)], dtype=np.int32)
_COLMAP = np.full(NP * NP, 351, dtype=np.int32)
_COLMAP[_LI * NP + _LJ] = np.arange(351, dtype=np.int32)


def kernel(dense_x, lS_o, lS_i, emb,
           W_bot0, b_bot0, W_bot1, b_bot1, W_bot2, b_bot2,
           W_top0, b_top0, W_top1, b_top1, W_top2, b_top2,
           W_top3, b_top3, W_top4, b_top4):
    gidx = lS_i + (jnp.arange(NF, dtype=jnp.int32) * V)[:, None]
    gidx4 = gidx.reshape(NF, NSPLIT, NW, CHUNK).transpose(1, 2, 0, 3)
    table2 = emb.reshape(NF * V, M)
    pooled_chunks = [_sc_gather(table2, gidx4[c]) for c in range(NSPLIT)]

    wsrc = jnp.concatenate(
        [W_top0[:, M:], jnp.zeros((W_top0.shape[0], 1), jnp.float32)], axis=1)
    wz2 = wsrc[:, _COLMAP]
    wx = W_top0[:, :M]

    args = [
        W_bot0, b_bot0.reshape(1, -1), W_bot1, b_bot1.reshape(1, -1),
        W_bot2, b_bot2.reshape(1, -1),
        wx.astype(jnp.bfloat16),
        wz2.astype(jnp.bfloat16).reshape(-1, 8, 4 * NP).transpose(1, 0, 2),
        b_top0.reshape(1, -1),
        W_top1.astype(jnp.bfloat16), b_top1.reshape(1, -1),
        W_top2.astype(jnp.bfloat16), b_top2.reshape(1, -1),
        W_top3, b_top3.reshape(1, -1),
        jnp.concatenate([W_top4, jnp.zeros((7, W_top4.shape[1]), jnp.float32)]),
        b_top4.reshape(1, -1),
    ]
    outs = [
        _tc_call(dense_x[c * BSPLIT:(c + 1) * BSPLIT], pooled_chunks[c], args)
        for c in range(NSPLIT)
    ]
    return jnp.concatenate(outs, axis=0)


# speedup vs baseline: 1.5903x; 1.0057x over previous
import functools

import jax
import jax.numpy as jnp
import numpy as np
from jax import lax
from jax.experimental import pallas as pl
from jax.experimental.pallas import tpu as pltpu
from jax.experimental.pallas import tpu_sc as plsc

B = 4096
NF = 26
V = 100000
M = 128
NW = 32
CHUNK = 64

NSPLIT = 2
BSPLIT = B // NSPLIT
BB = 512
GRID = BSPLIT // BB
NI = 27
NP = 32


def _sc_gather(table2, gidx3):
    mesh = plsc.VectorSubcoreMesh(core_axis_name="c", subcore_axis_name="s")

    @functools.partial(
        pl.kernel,
        out_type=jax.ShapeDtypeStruct((BSPLIT, NP, M), jnp.float32),
        mesh=mesh,
        scratch_types=[
            pltpu.VMEM((NF, CHUNK), jnp.int32),
            pltpu.VMEM((CHUNK, 1, M), jnp.float32),
            pltpu.SemaphoreType.DMA,
        ],
    )
    def k(table_hbm, gidx_hbm, out_hbm, idx_v, rows_v, sem):
        w = lax.axis_index("s") * 2 + lax.axis_index("c")
        b0 = w * CHUNK
        pltpu.sync_copy(gidx_hbm.at[w], idx_v)

        def body(kf, carry):
            pltpu.async_copy(table_hbm.at[idx_v.at[kf]], rows_v.at[:, 0], sem).wait()
            pltpu.sync_copy(rows_v,
                            out_hbm.at[pl.ds(b0, CHUNK), pl.ds(kf + 1, 1)])
            return carry

        lax.fori_loop(0, NF, body, 0)

    return k(table2, gidx3)


def _dott(a, w):
    return lax.dot_general(a, w, (((1,), (1,)), ((), ())),
                           preferred_element_type=jnp.float32)


def _tc_body(xd_ref, pooled_ref,
             wb0, bb0, wb1, bb1, wb2, bb2,
             wx, wz2, bt0, wt1, bt1, wt2, bt2, wt3, bt3, wt4, bt4,
             out_ref, tpad, zf3):
    h = jnp.maximum(_dott(xd_ref[...], wb0[...]) + bb0[...], 0.0)
    h = jnp.maximum(_dott(h, wb1[...]) + bb1[...], 0.0)
    xb = jnp.maximum(_dott(h, wb2[...]) + bb2[...], 0.0)

    tpad[...] = pooled_ref[...].astype(jnp.bfloat16)
    tpad[:, 0:1, :] = xb[:, None, :].astype(jnp.bfloat16)
    tpad[:, NI:NP, :] = jnp.zeros((BB, NP - NI, M), jnp.bfloat16)

    for g in range(BB // 8):
        ts = tpad[g * 8:(g + 1) * 8].reshape(8 * NP, M)
        zs = lax.dot_general(ts, ts, (((1,), (1,)), ((), ())),
                             preferred_element_type=jnp.float32)
        for s in range(8):
            zf3[g * 8 + s, :, :] = zs[s * NP:(s + 1) * NP,
                                      s * NP:(s + 1) * NP].astype(jnp.bfloat16)

    zint = jnp.zeros((BB, 1024), jnp.float32)
    for q in range(NP // 4):
        zq = zf3[:, 4 * q:4 * (q + 1), :].reshape(BB, 4 * NP)
        zint = zint + _dott(zq, wz2[q])

    l0 = jnp.maximum(
        _dott(xb.astype(jnp.bfloat16), wx[...]) + zint + bt0[...], 0.0)
    l1 = jnp.maximum(_dott(l0.astype(jnp.bfloat16), wt1[...]) + bt1[...], 0.0)
    l2 = jnp.maximum(_dott(l1.astype(jnp.bfloat16), wt2[...]) + bt2[...], 0.0)
    l3 = jnp.maximum(_dott(l2, wt3[...]) + bt3[...], 0.0)
    l4 = _dott(l3, wt4[...])[:, 0:1]
    out_ref[...] = jax.nn.sigmoid(l4 + bt4[0, 0])


def _tc_call(dense_x, pooled3, args):
    full = lambda shape: pl.BlockSpec(shape, lambda i: (0,) * len(shape))
    in_specs = [
        pl.BlockSpec((BB, 13), lambda i: (i, 0)),
        pl.BlockSpec((BB, NP, M), lambda i: (i, 0, 0)),
    ] + [full(a.shape) for a in args[:-1]] + [
        pl.BlockSpec(memory_space=pltpu.SMEM)]
    return pl.pallas_call(
        _tc_body,
        grid=(GRID,),
        in_specs=in_specs,
        out_specs=pl.BlockSpec((BB, 1), lambda i: (i, 0)),
        out_shape=jax.ShapeDtypeStruct((BSPLIT, 1), jnp.float32),
        scratch_shapes=[
            pltpu.VMEM((BB, NP, M), jnp.bfloat16),
            pltpu.VMEM((BB, NP, NP), jnp.bfloat16),
        ],
        compiler_params=pltpu.CompilerParams(
            dimension_semantics=("arbitrary",)),
    )(dense_x, pooled3, *args)


_LI = np.array([i for i in range(NI) for j in range(i)], dtype=np.int32)
_LJ = np.array([j for i in range(NI) for j in range(i)], dtype=np.int32)
_COLMAP = np.full(NP * NP, 351, dtype=np.int32)
_COLMAP[_LI * NP + _LJ] = np.arange(351, dtype=np.int32)


def kernel(dense_x, lS_o, lS_i, emb,
           W_bot0, b_bot0, W_bot1, b_bot1, W_bot2, b_bot2,
           W_top0, b_top0, W_top1, b_top1, W_top2, b_top2,
           W_top3, b_top3, W_top4, b_top4):
    gidx = lS_i + (jnp.arange(NF, dtype=jnp.int32) * V)[:, None]
    gidx4 = gidx.reshape(NF, NSPLIT, NW, CHUNK).transpose(1, 2, 0, 3)
    table2 = emb.reshape(NF * V, M)
    pooled_chunks = [_sc_gather(table2, gidx4[c]) for c in range(NSPLIT)]

    wsrc = jnp.concatenate(
        [W_top0[:, M:], jnp.zeros((W_top0.shape[0], 1), jnp.float32)], axis=1)
    wz2 = wsrc[:, _COLMAP]
    wx = W_top0[:, :M]

    args = [
        W_bot0, b_bot0.reshape(1, -1), W_bot1, b_bot1.reshape(1, -1),
        W_bot2, b_bot2.reshape(1, -1),
        wx.astype(jnp.bfloat16),
        wz2.astype(jnp.bfloat16).reshape(-1, 8, 4 * NP).transpose(1, 0, 2),
        b_top0.reshape(1, -1),
        W_top1.astype(jnp.bfloat16), b_top1.reshape(1, -1),
        W_top2.astype(jnp.bfloat16), b_top2.reshape(1, -1),
        W_top3, b_top3.reshape(1, -1),
        jnp.concatenate([W_top4, jnp.zeros((7, W_top4.shape[1]), jnp.float32)]),
        b_top4.reshape(1, -1),
    ]
    outs = [
        _tc_call(dense_x[c * BSPLIT:(c + 1) * BSPLIT], pooled_chunks[c], args)
        for c in range(NSPLIT)
    ]
    return jnp.concatenate(outs, axis=0)


# speedup vs baseline: 1.7834x; 1.1214x over previous
import functools

import jax
import jax.numpy as jnp
import numpy as np
from jax import lax
from jax.experimental import pallas as pl
from jax.experimental.pallas import tpu as pltpu
from jax.experimental.pallas import tpu_sc as plsc

B = 4096
NF = 26
V = 100000
M = 128
NW = 32
CHUNK = 64

NSPLIT = 2
BSPLIT = B // NSPLIT
BB = 512
GRID = BSPLIT // BB
NI = 27
NP = 32


def _sc_gather(table2, gidx3):
    mesh = plsc.VectorSubcoreMesh(core_axis_name="c", subcore_axis_name="s")

    @functools.partial(
        pl.kernel,
        out_type=jax.ShapeDtypeStruct((BSPLIT, NP, M), jnp.float32),
        mesh=mesh,
        scratch_types=[
            pltpu.VMEM((NF, CHUNK), jnp.int32),
            pltpu.VMEM((CHUNK, 1, M), jnp.float32),
            pltpu.VMEM((CHUNK, 1, M), jnp.float32),
            pltpu.VMEM((CHUNK, 1, M), jnp.float32),
            pltpu.VMEM((CHUNK, 1, M), jnp.float32),
            pltpu.SemaphoreType.DMA,
            pltpu.SemaphoreType.DMA,
            pltpu.SemaphoreType.DMA,
            pltpu.SemaphoreType.DMA,
        ],
    )
    def k(table_hbm, gidx_hbm, out_hbm, idx_v,
          rb00, rb01, rb10, rb11, sg0, sg1, sw0, sw1):
        w = lax.axis_index("s") * 2 + lax.axis_index("c")
        b0 = w * CHUNK
        pltpu.sync_copy(gidx_hbm.at[w], idx_v)

        rbs = ((rb00, rb01), (rb10, rb11))
        gsems, wsems = (sg0, sg1), (sw0, sw1)
        nr = NF // 2
        gh, wh = {}, {}

        def issue(kk):
            for f in range(2):
                gh[(kk, f)] = pltpu.async_copy(
                    table_hbm.at[idx_v.at[2 * kk + f]],
                    rbs[kk % 2][f].at[:, 0], gsems[kk % 2])

        issue(0)
        for kk in range(nr):
            gh[(kk, 0)].wait()
            gh[(kk, 1)].wait()
            if kk + 1 < nr:
                if kk >= 1:
                    wh[(kk - 1, 0)].wait()
                    wh[(kk - 1, 1)].wait()
                issue(kk + 1)
            for f in range(2):
                wh[(kk, f)] = pltpu.async_copy(
                    rbs[kk % 2][f],
                    out_hbm.at[pl.ds(b0, CHUNK), pl.ds(2 * kk + 1 + f, 1)],
                    wsems[kk % 2])
        for f in range(2):
            wh[(nr - 2, f)].wait()
            wh[(nr - 1, f)].wait()

    return k(table2, gidx3)


def _dott(a, w):
    return lax.dot_general(a, w, (((1,), (1,)), ((), ())),
                           preferred_element_type=jnp.float32)


def _tc_body(xd_ref, pooled_ref,
             wb0, bb0, wb1, bb1, wb2, bb2,
             wx, wz2, bt0, wt1, bt1, wt2, bt2, wt3, bt3, wt4, bt4,
             out_ref, tpad, zf3):
    h = jnp.maximum(_dott(xd_ref[...], wb0[...]) + bb0[...], 0.0)
    h = jnp.maximum(_dott(h, wb1[...]) + bb1[...], 0.0)
    xb = jnp.maximum(_dott(h, wb2[...]) + bb2[...], 0.0)

    tpad[...] = pooled_ref[...].astype(jnp.bfloat16)
    tpad[:, 0:1, :] = xb[:, None, :].astype(jnp.bfloat16)
    tpad[:, NI:NP, :] = jnp.zeros((BB, NP - NI, M), jnp.bfloat16)

    for g in range(BB // 8):
        ts = tpad[g * 8:(g + 1) * 8].reshape(8 * NP, M)
        zs = lax.dot_general(ts, ts, (((1,), (1,)), ((), ())),
                             preferred_element_type=jnp.float32)
        for s in range(8):
            zf3[g * 8 + s, :, :] = zs[s * NP:(s + 1) * NP,
                                      s * NP:(s + 1) * NP].astype(jnp.bfloat16)

    zint = jnp.zeros((BB, 1024), jnp.float32)
    for q in range(NP // 4):
        zq = zf3[:, 4 * q:4 * (q + 1), :].reshape(BB, 4 * NP)
        zint = zint + _dott(zq, wz2[q])

    l0 = jnp.maximum(
        _dott(xb.astype(jnp.bfloat16), wx[...]) + zint + bt0[...], 0.0)
    l1 = jnp.maximum(_dott(l0.astype(jnp.bfloat16), wt1[...]) + bt1[...], 0.0)
    l2 = jnp.maximum(_dott(l1.astype(jnp.bfloat16), wt2[...]) + bt2[...], 0.0)
    l3 = jnp.maximum(_dott(l2, wt3[...]) + bt3[...], 0.0)
    l4 = _dott(l3, wt4[...])[:, 0:1]
    out_ref[...] = jax.nn.sigmoid(l4 + bt4[0, 0])


def _tc_call(dense_x, pooled3, args):
    full = lambda shape: pl.BlockSpec(shape, lambda i: (0,) * len(shape))
    in_specs = [
        pl.BlockSpec((BB, 13), lambda i: (i, 0)),
        pl.BlockSpec((BB, NP, M), lambda i: (i, 0, 0)),
    ] + [full(a.shape) for a in args[:-1]] + [
        pl.BlockSpec(memory_space=pltpu.SMEM)]
    return pl.pallas_call(
        _tc_body,
        grid=(GRID,),
        in_specs=in_specs,
        out_specs=pl.BlockSpec((BB, 1), lambda i: (i, 0)),
        out_shape=jax.ShapeDtypeStruct((BSPLIT, 1), jnp.float32),
        scratch_shapes=[
            pltpu.VMEM((BB, NP, M), jnp.bfloat16),
            pltpu.VMEM((BB, NP, NP), jnp.bfloat16),
        ],
        compiler_params=pltpu.CompilerParams(
            dimension_semantics=("arbitrary",)),
    )(dense_x, pooled3, *args)


_LI = np.array([i for i in range(NI) for j in range(i)], dtype=np.int32)
_LJ = np.array([j for i in range(NI) for j in range(i)], dtype=np.int32)
_COLMAP = np.full(NP * NP, 351, dtype=np.int32)
_COLMAP[_LI * NP + _LJ] = np.arange(351, dtype=np.int32)


def kernel(dense_x, lS_o, lS_i, emb,
           W_bot0, b_bot0, W_bot1, b_bot1, W_bot2, b_bot2,
           W_top0, b_top0, W_top1, b_top1, W_top2, b_top2,
           W_top3, b_top3, W_top4, b_top4):
    gidx = lS_i + (jnp.arange(NF, dtype=jnp.int32) * V)[:, None]
    gidx4 = gidx.reshape(NF, NSPLIT, NW, CHUNK).transpose(1, 2, 0, 3)
    table2 = emb.reshape(NF * V, M)
    pooled_chunks = [_sc_gather(table2, gidx4[c]) for c in range(NSPLIT)]

    wsrc = jnp.concatenate(
        [W_top0[:, M:], jnp.zeros((W_top0.shape[0], 1), jnp.float32)], axis=1)
    wz2 = wsrc[:, _COLMAP]
    wx = W_top0[:, :M]

    args = [
        W_bot0, b_bot0.reshape(1, -1), W_bot1, b_bot1.reshape(1, -1),
        W_bot2, b_bot2.reshape(1, -1),
        wx.astype(jnp.bfloat16),
        wz2.astype(jnp.bfloat16).reshape(-1, 8, 4 * NP).transpose(1, 0, 2),
        b_top0.reshape(1, -1),
        W_top1.astype(jnp.bfloat16), b_top1.reshape(1, -1),
        W_top2.astype(jnp.bfloat16), b_top2.reshape(1, -1),
        W_top3, b_top3.reshape(1, -1),
        jnp.concatenate([W_top4, jnp.zeros((7, W_top4.shape[1]), jnp.float32)]),
        b_top4.reshape(1, -1),
    ]
    outs = [
        _tc_call(dense_x[c * BSPLIT:(c + 1) * BSPLIT], pooled_chunks[c], args)
        for c in range(NSPLIT)
    ]
    return jnp.concatenate(outs, axis=0)
